# baseline (speedup 1.0000x reference)
import jax
import jax.numpy as jnp
from jax import lax
from jax.experimental import pallas as pl

BMH = 8192
KS = 200


def _matmul_block(xt_ref, wt_ref, o_ref):
    j = pl.program_id(1)
    part = lax.dot_general(
        xt_ref[...], wt_ref[...],
        dimension_numbers=(((0,), (0,)), ((), ())),
        preferred_element_type=jnp.float32,
    )

    @pl.when(j == 0)
    def _init():
        o_ref[...] = part

    @pl.when(j > 0)
    def _acc():
        o_ref[...] += part


@jax.jit
def kernel(x, W):
    B, K = x.shape
    N = W.shape[0]
    xt = x.T
    wt = W.T
    grid = (B // BMH, K // KS)
    return pl.pallas_call(
        _matmul_block,
        grid=grid,
        in_specs=[
            pl.BlockSpec((KS, BMH), lambda i, j: (j, i)),
            pl.BlockSpec((KS, N), lambda i, j: (j, 0)),
        ],
        out_specs=pl.BlockSpec((BMH, N), lambda i, j: (i, 0)),
        out_shape=jax.ShapeDtypeStruct((B, N), jnp.float32),
    )(xt, wt)


# speedup vs baseline: 1.1884x; 1.1884x over previous
import jax
import jax.numpy as jnp
from jax import lax
from jax.experimental import pallas as pl

BM = 2048


def _matmul_block(xt_ref, wt_ref, o_ref):
    o_ref[...] = lax.dot_general(
        xt_ref[...], wt_ref[...],
        dimension_numbers=(((0,), (0,)), ((), ())),
        preferred_element_type=jnp.float32,
    )


@jax.jit
def kernel(x, W):
    B, K = x.shape
    N = W.shape[0]
    xt = x.T
    wt = W.T
    grid = (B // BM,)
    return pl.pallas_call(
        _matmul_block,
        grid=grid,
        in_specs=[
            pl.BlockSpec((K, BM), lambda i: (0, i)),
            pl.BlockSpec((K, N), lambda i: (0, 0)),
        ],
        out_specs=pl.BlockSpec((BM, N), lambda i: (i, 0)),
        out_shape=jax.ShapeDtypeStruct((B, N), jnp.float32),
    )(xt, wt)
